# baseline (speedup 1.0000x reference)
import jax
import jax.numpy as jnp
from jax.experimental import pallas as pl
import jax.experimental.pallas.tpu as pltpu

E = 16
K = 2
H = 768
FF = 768
LIMIT = 7.0
EPB = 2
STEPS = E // EPB


def _moe_body(x_ref, w1_ref, b1_ref, w2_ref, b2_ref, rw_ref, rb_ref,
              out_ref, aux_ref, g_ref):
    e = pl.program_id(0)
    n_tok = x_ref.shape[0]
    x = x_ref[...]
    iota = jax.lax.broadcasted_iota(jnp.int32, (n_tok, E), 1)

    @pl.when(e == 0)
    def _router():
        logits = jnp.dot(x, rw_ref[...].T,
                         preferred_element_type=jnp.float32) + rb_ref[...]
        m = jnp.max(logits, axis=-1, keepdims=True)
        ex = jnp.exp(logits - m)
        probs = ex / jnp.sum(ex, axis=-1, keepdims=True)
        m1 = jnp.max(probs, axis=-1, keepdims=True)
        i1 = jnp.min(jnp.where(probs == m1, iota, E), axis=-1, keepdims=True)
        masked = jnp.where(iota == i1, -jnp.inf, probs)
        m2 = jnp.max(masked, axis=-1, keepdims=True)
        i2 = jnp.min(jnp.where(masked == m2, iota, E), axis=-1, keepdims=True)
        denom = m1 + m2 + 1e-9
        g = jnp.where(iota == i1, m1 / denom,
                      jnp.where(iota == i2, m2 / denom, 0.0))
        g_ref[...] = g
        importance = jnp.mean(probs, axis=0, keepdims=True)
        load = jnp.mean((iota == i1).astype(jnp.float32), axis=0,
                        keepdims=True)
        aux_ref[...] = jnp.sum(E * importance * load).reshape(1, 1)

    g = g_ref[...]
    acc = jnp.zeros((n_tok, H), jnp.float32)
    for ei in range(EPB):
        u = jnp.dot(x, w1_ref[ei],
                    preferred_element_type=jnp.float32) + b1_ref[ei]
        up = jnp.clip(u[:, :FF], -LIMIT, LIMIT)
        gate = jnp.clip(u[:, FF:], -LIMIT, LIMIT)
        act = gate * jax.nn.sigmoid(gate) * up
        z = jnp.dot(act, w2_ref[ei],
                    preferred_element_type=jnp.float32) + b2_ref[ei]
        ge = jnp.sum(jnp.where(iota == EPB * e + ei, g, 0.0), axis=-1,
                     keepdims=True)
        acc = acc + ge * z

    @pl.when(e == 0)
    def _init():
        out_ref[...] = acc

    @pl.when(e > 0)
    def _acc():
        out_ref[...] = out_ref[...] + acc


def kernel(x, ffn_in, ffn_in_bias, ffn_out, ffn_out_bias, router_w, router_b):
    b, t, h = x.shape
    n_tok = b * t
    x2d = x.reshape(n_tok, h)
    rb2d = router_b.reshape(1, E)
    b1_3d = ffn_in_bias.reshape(E, 1, 2 * FF)
    b2_3d = ffn_out_bias.reshape(E, 1, H)

    out2d, aux = pl.pallas_call(
        _moe_body,
        grid=(STEPS,),
        in_specs=[
            pl.BlockSpec((n_tok, H), lambda e: (0, 0)),
            pl.BlockSpec((EPB, H, 2 * FF), lambda e: (e, 0, 0)),
            pl.BlockSpec((EPB, 1, 2 * FF), lambda e: (e, 0, 0)),
            pl.BlockSpec((EPB, FF, H), lambda e: (e, 0, 0)),
            pl.BlockSpec((EPB, 1, H), lambda e: (e, 0, 0)),
            pl.BlockSpec((E, H), lambda e: (0, 0)),
            pl.BlockSpec((1, E), lambda e: (0, 0)),
        ],
        out_specs=[
            pl.BlockSpec((n_tok, H), lambda e: (0, 0)),
            pl.BlockSpec((1, 1), lambda e: (0, 0)),
        ],
        out_shape=[
            jax.ShapeDtypeStruct((n_tok, H), jnp.float32),
            jax.ShapeDtypeStruct((1, 1), jnp.float32),
        ],
        scratch_shapes=[pltpu.VMEM((n_tok, E), jnp.float32)],
        compiler_params=pltpu.CompilerParams(
            dimension_semantics=("arbitrary",),
        ),
    )(x2d, ffn_in, b1_3d, ffn_out, b2_3d, router_w, rb2d)

    return out2d.reshape(b, t, h), aux.reshape(())


# speedup vs baseline: 1.0861x; 1.0861x over previous
import jax
import jax.numpy as jnp
from jax.experimental import pallas as pl
import jax.experimental.pallas.tpu as pltpu

E = 16
K = 2
H = 768
FF = 768
LIMIT = 7.0


def _moe_body(x_ref, w1_ref, b1_ref, w2_ref, b2_ref, rw_ref, rb_ref,
              out_ref, aux_ref, g_ref):
    e = pl.program_id(0)
    n_tok = x_ref.shape[0]
    x = x_ref[...]
    iota = jax.lax.broadcasted_iota(jnp.int32, (n_tok, E), 1)

    @pl.when(e == 0)
    def _router():
        logits = jnp.dot(x, rw_ref[...].T,
                         preferred_element_type=jnp.float32) + rb_ref[...]
        m = jnp.max(logits, axis=-1, keepdims=True)
        ex = jnp.exp(logits - m)
        probs = ex / jnp.sum(ex, axis=-1, keepdims=True)
        m1 = jnp.max(probs, axis=-1, keepdims=True)
        i1 = jnp.min(jnp.where(probs == m1, iota, E), axis=-1, keepdims=True)
        masked = jnp.where(iota == i1, -jnp.inf, probs)
        m2 = jnp.max(masked, axis=-1, keepdims=True)
        i2 = jnp.min(jnp.where(masked == m2, iota, E), axis=-1, keepdims=True)
        denom = m1 + m2 + 1e-9
        g = jnp.where(iota == i1, m1 / denom,
                      jnp.where(iota == i2, m2 / denom, 0.0))
        g_ref[...] = g
        importance = jnp.mean(probs, axis=0, keepdims=True)
        load = jnp.mean((iota == i1).astype(jnp.float32), axis=0,
                        keepdims=True)
        aux_ref[...] = jnp.sum(E * importance * load).reshape(1, 1)

    u = jnp.dot(x, w1_ref[0], preferred_element_type=jnp.float32) + b1_ref[e]
    up = jnp.clip(u[:, :FF], -LIMIT, LIMIT)
    gate = jnp.clip(u[:, FF:], -LIMIT, LIMIT)
    act = gate * jax.nn.sigmoid(gate) * up
    z = jnp.dot(act, w2_ref[0], preferred_element_type=jnp.float32) + b2_ref[e]
    ge = jnp.sum(jnp.where(iota == e, g_ref[...], 0.0), axis=-1,
                 keepdims=True)
    contrib = ge * z

    @pl.when(e == 0)
    def _init():
        out_ref[...] = contrib

    @pl.when(e > 0)
    def _acc():
        out_ref[...] = out_ref[...] + contrib


def kernel(x, ffn_in, ffn_in_bias, ffn_out, ffn_out_bias, router_w, router_b):
    b, t, h = x.shape
    n_tok = b * t
    x2d = x.reshape(n_tok, h)
    rb2d = router_b.reshape(1, E)
    b1_3d = ffn_in_bias.reshape(E, 1, 2 * FF)
    b2_3d = ffn_out_bias.reshape(E, 1, H)

    out2d, aux = pl.pallas_call(
        _moe_body,
        grid=(E,),
        in_specs=[
            pl.BlockSpec((n_tok, H), lambda e: (0, 0)),
            pl.BlockSpec((1, H, 2 * FF), lambda e: (e, 0, 0)),
            pl.BlockSpec((E, 1, 2 * FF), lambda e: (0, 0, 0)),
            pl.BlockSpec((1, FF, H), lambda e: (e, 0, 0)),
            pl.BlockSpec((E, 1, H), lambda e: (0, 0, 0)),
            pl.BlockSpec((E, H), lambda e: (0, 0)),
            pl.BlockSpec((1, E), lambda e: (0, 0)),
        ],
        out_specs=[
            pl.BlockSpec((n_tok, H), lambda e: (0, 0)),
            pl.BlockSpec((1, 1), lambda e: (0, 0)),
        ],
        out_shape=[
            jax.ShapeDtypeStruct((n_tok, H), jnp.float32),
            jax.ShapeDtypeStruct((1, 1), jnp.float32),
        ],
        scratch_shapes=[pltpu.VMEM((n_tok, E), jnp.float32)],
        compiler_params=pltpu.CompilerParams(
            dimension_semantics=("arbitrary",),
        ),
    )(x2d, ffn_in, b1_3d, ffn_out, b2_3d, router_w, rb2d)

    return out2d.reshape(b, t, h), aux.reshape(())
